# baseline (speedup 1.0000x reference)
import jax
import jax.numpy as jnp
from jax.experimental import pallas as pl
from jax.experimental.pallas import tpu as pltpu

N = 4096
C = 128
IN = 256
HOPS = 4
TH = 0.6
T = 0.07

BI = 512
BJ = 512
GI = N // BI
GJ = N // BJ


def _embed_body(x_ref, w_ref, b_ref, o_ref):
    y = jnp.dot(x_ref[...], w_ref[...], preferred_element_type=jnp.float32)
    y = y + b_ref[...]
    nrm = jnp.sqrt(jnp.sum(y * y, axis=1, keepdims=True))
    o_ref[...] = y / nrm


def _adj_body(ei_ref, ej_ref, a_ref, norm_ref):
    i = pl.program_id(0)
    j = pl.program_id(1)
    d = jax.lax.dot_general(ei_ref[...], ej_ref[...], (((1,), (1,)), ((), ())),
                            preferred_element_type=jnp.float32)
    rowid = jax.lax.broadcasted_iota(jnp.int32, (BI, BJ), 0)
    colid = jax.lax.broadcasted_iota(jnp.int32, (BI, BJ), 1)
    eye = (rowid == colid) & (i == j)
    a = (d > TH).astype(jnp.float32) + eye.astype(jnp.float32)
    a_ref[...] = a.astype(jnp.int8)
    deg = jnp.sum(a, axis=1, keepdims=True)

    @pl.when(j == 0)
    def _():
        norm_ref[...] = deg

    @pl.when(j > 0)
    def _():
        norm_ref[...] += deg

    @pl.when(j == GJ - 1)
    def _():
        norm_ref[...] = jax.lax.rsqrt(jnp.clip(norm_ref[...], 1.0, None))


def _hop_body(a_ref, h_ref, ni_ref, nj_ref, o_ref):
    j = pl.program_id(1)
    hn = h_ref[...] * nj_ref[...]
    p = jnp.dot(a_ref[...].astype(jnp.float32), hn,
                preferred_element_type=jnp.float32)

    @pl.when(j == 0)
    def _():
        o_ref[...] = p

    @pl.when(j > 0)
    def _():
        o_ref[...] += p

    @pl.when(j == GJ - 1)
    def _():
        o_ref[...] *= ni_ref[...]


def _tag_body(h0_ref, h1_ref, h2_ref, h3_ref, h4_ref, w_ref, b_ref,
              gt_ref, gs_ref):
    w = w_ref[...]
    b = b_ref[...]
    hs = (h0_ref[...], h1_ref[...], h2_ref[...], h3_ref[...], h4_ref[...])
    for col, out in ((0, gt_ref), (C, gs_ref)):
        raw = b
        for k in range(HOPS + 1):
            raw = raw + jnp.dot(hs[k][:, col:col + C], w[k * C:(k + 1) * C, :],
                                preferred_element_type=jnp.float32)
        nrm = jnp.sqrt(jnp.sum(raw * raw, axis=1, keepdims=True))
        out[...] = raw / nrm


def _loss_body(gt_ref, gs_ref, o_ref, m_ref, s_ref, p_ref):
    i = pl.program_id(0)
    j = pl.program_id(1)
    tile = jax.lax.dot_general(gt_ref[...], gs_ref[...], (((1,), (1,)), ((), ())),
                               preferred_element_type=jnp.float32) * (1.0 / T)

    @pl.when(i == j)
    def _():
        p_ref[...] = jnp.sum(gt_ref[...] * gs_ref[...], axis=1,
                             keepdims=True) * (1.0 / T)

    rowid = jax.lax.broadcasted_iota(jnp.int32, (BI, BJ), 0)
    colid = jax.lax.broadcasted_iota(jnp.int32, (BI, BJ), 1)
    diag = (rowid == colid) & (i == j)
    tile = jnp.where(diag, -10.0 / T, tile)

    @pl.when(j == 0)
    def _():
        m_ref[...] = jnp.full((BI, 1), -jnp.inf, jnp.float32)
        s_ref[...] = jnp.zeros((BI, 1), jnp.float32)

    m_prev = m_ref[...]
    s_prev = s_ref[...]
    m_new = jnp.maximum(m_prev, jnp.max(tile, axis=1, keepdims=True))
    s_new = s_prev * jnp.exp(m_prev - m_new) + jnp.sum(
        jnp.exp(tile - m_new), axis=1, keepdims=True)
    m_ref[...] = m_new
    s_ref[...] = s_new

    @pl.when((i == 0) & (j == 0))
    def _():
        o_ref[...] = jnp.zeros((1, 1), jnp.float32)

    @pl.when(j == GJ - 1)
    def _():
        pos = p_ref[...]
        mm = jnp.maximum(m_ref[...], pos)
        lse = mm + jnp.log(s_ref[...] * jnp.exp(m_ref[...] - mm)
                           + jnp.exp(pos - mm))
        contrib = jnp.sum(lse - pos)
        tot = o_ref[...] + contrib
        o_ref[...] = jnp.where(i == GI - 1, tot * (1.0 / N), tot)


def kernel(feat_s, feat_t, W_embed, b_embed, W_tag, b_tag):
    b_embed2 = b_embed.reshape(1, C)
    b_tag2 = b_tag.reshape(1, C)

    feats = jnp.concatenate([feat_t, feat_s], axis=0)
    emb = pl.pallas_call(
        _embed_body,
        grid=(2 * GI,),
        in_specs=[
            pl.BlockSpec((BI, IN), lambda i: (i, 0)),
            pl.BlockSpec((IN, C), lambda i: (0, 0)),
            pl.BlockSpec((1, C), lambda i: (0, 0)),
        ],
        out_specs=pl.BlockSpec((BI, C), lambda i: (i, 0)),
        out_shape=jax.ShapeDtypeStruct((2 * N, C), jnp.float32),
    )(feats, W_embed, b_embed2)
    f_et = emb[:N]
    f_es = emb[N:]

    adj, norm = pl.pallas_call(
        _adj_body,
        grid=(GI, GJ),
        in_specs=[
            pl.BlockSpec((BI, C), lambda i, j: (i, 0)),
            pl.BlockSpec((BJ, C), lambda i, j: (j, 0)),
        ],
        out_specs=[
            pl.BlockSpec((BI, BJ), lambda i, j: (i, j)),
            pl.BlockSpec((BI, 1), lambda i, j: (i, 0)),
        ],
        out_shape=[
            jax.ShapeDtypeStruct((N, N), jnp.int8),
            jax.ShapeDtypeStruct((N, 1), jnp.float32),
        ],
    )(f_et, f_et)

    hop_call = pl.pallas_call(
        _hop_body,
        grid=(GI, GJ),
        in_specs=[
            pl.BlockSpec((BI, BJ), lambda i, j: (i, j)),
            pl.BlockSpec((BJ, 2 * C), lambda i, j: (j, 0)),
            pl.BlockSpec((BI, 1), lambda i, j: (i, 0)),
            pl.BlockSpec((BJ, 1), lambda i, j: (j, 0)),
        ],
        out_specs=pl.BlockSpec((BI, 2 * C), lambda i, j: (i, 0)),
        out_shape=jax.ShapeDtypeStruct((N, 2 * C), jnp.float32),
    )

    hs = [jnp.concatenate([f_et, f_es], axis=1)]
    for _ in range(HOPS):
        hs.append(hop_call(adj, hs[-1], norm, norm))

    f_gt, f_gs = pl.pallas_call(
        _tag_body,
        grid=(GI,),
        in_specs=[pl.BlockSpec((BI, 2 * C), lambda i: (i, 0))] * (HOPS + 1)
        + [
            pl.BlockSpec(((HOPS + 1) * C, C), lambda i: (0, 0)),
            pl.BlockSpec((1, C), lambda i: (0, 0)),
        ],
        out_specs=[
            pl.BlockSpec((BI, C), lambda i: (i, 0)),
            pl.BlockSpec((BI, C), lambda i: (i, 0)),
        ],
        out_shape=[
            jax.ShapeDtypeStruct((N, C), jnp.float32),
            jax.ShapeDtypeStruct((N, C), jnp.float32),
        ],
    )(*hs, W_tag, b_tag2)

    loss = pl.pallas_call(
        _loss_body,
        grid=(GI, GJ),
        in_specs=[
            pl.BlockSpec((BI, C), lambda i, j: (i, 0)),
            pl.BlockSpec((BJ, C), lambda i, j: (j, 0)),
        ],
        out_specs=pl.BlockSpec((1, 1), lambda i, j: (0, 0)),
        out_shape=jax.ShapeDtypeStruct((1, 1), jnp.float32),
        scratch_shapes=[
            pltpu.VMEM((BI, 1), jnp.float32),
            pltpu.VMEM((BI, 1), jnp.float32),
            pltpu.VMEM((BI, 1), jnp.float32),
        ],
    )(f_gt, f_gs)

    return loss.reshape(())
